# Initial kernel scaffold; baseline (speedup 1.0000x reference)
import jax
import jax.numpy as jnp
from jax.experimental import pallas as pl


def kernel(x, edge_index, W1, b1, W2, b2, LW1, Lb1, LW2, Lb2):
    raise NotImplementedError("write your pallas kernel here")


# speedup vs baseline: 25.4288x; 25.4288x over previous
import functools

import jax
import jax.numpy as jnp
from jax import lax
from jax.experimental import pallas as pl
from jax.experimental.pallas import tpu as pltpu
from jax.experimental.pallas import tpu_sc as plsc

N = 10000
NPAD = 10240
E = 320000
DIN = 128
DH = 20
DOUT = 128
F = 48
NC, NS = 2, 16
NTILES = NC * NS
CHUNK = 128
EPT = 10112
EPAD = NTILES * EPT
NCHUNK = EPT // CHUNK
SLICE = NPAD // NS


def _sc_degree(dst_hbm, ones_hbm, zeros_hbm, cnt_hbm, idx_v, ones_v, stage_v,
               cnt_sh):
    c = lax.axis_index("c")
    s = lax.axis_index("s")
    tid = c * NS + s
    pltpu.sync_copy(zeros_hbm.at[pl.ds(s * SLICE, SLICE)], stage_v)
    pltpu.sync_copy(stage_v, cnt_sh.at[pl.ds(s * SLICE, SLICE)])
    pltpu.sync_copy(ones_hbm, ones_v)
    plsc.subcore_barrier()

    def body(i, carry):
        base = tid * EPT + i * CHUNK
        pltpu.sync_copy(dst_hbm.at[pl.ds(base, CHUNK)], idx_v)
        pltpu.sync_copy(ones_v, cnt_sh.at[idx_v], add=True)
        return carry

    lax.fori_loop(0, NCHUNK, body, 0)
    plsc.subcore_barrier()
    pltpu.sync_copy(cnt_sh.at[pl.ds(s * SLICE, SLICE)], stage_v)
    pltpu.sync_copy(stage_v, cnt_hbm.at[c, pl.ds(s * SLICE, SLICE)])


@functools.cache
def _sc_degree_call():
    mesh = plsc.VectorSubcoreMesh(core_axis_name="c", subcore_axis_name="s",
                                  num_cores=NC, num_subcores=NS)
    return pl.kernel(
        _sc_degree,
        out_type=jax.ShapeDtypeStruct((NC, NPAD), jnp.float32),
        mesh=mesh,
        scratch_types=[
            pltpu.VMEM((CHUNK,), jnp.int32),
            pltpu.VMEM((CHUNK,), jnp.float32),
            pltpu.VMEM((SLICE,), jnp.float32),
            pltpu.VMEM_SHARED((NPAD,), jnp.float32),
        ],
        compiler_params=pltpu.CompilerParams(use_tc_tiling_on_sc=False),
    )


def _sc_aggregate(src_hbm, dst_hbm, y_hbm, zeros_hbm, agg_hbm, isrc_v, idst_v,
                  rows_v, stage_v, sem, agg_sh):
    c = lax.axis_index("c")
    s = lax.axis_index("s")
    tid = c * NS + s
    pltpu.sync_copy(zeros_hbm.at[pl.ds(s * SLICE, SLICE)], stage_v)
    pltpu.sync_copy(stage_v, agg_sh.at[pl.ds(s * SLICE, SLICE)])
    plsc.subcore_barrier()

    def body(i, carry):
        base = tid * EPT + i * CHUNK
        pltpu.sync_copy(src_hbm.at[pl.ds(base, CHUNK)], isrc_v)
        pltpu.sync_copy(dst_hbm.at[pl.ds(base, CHUNK)], idst_v)
        pltpu.async_copy(y_hbm.at[isrc_v], rows_v, sem).wait()
        pltpu.sync_copy(rows_v, agg_sh.at[idst_v], add=True)
        return carry

    lax.fori_loop(0, NCHUNK, body, 0)
    plsc.subcore_barrier()
    pltpu.sync_copy(agg_sh.at[pl.ds(s * SLICE, SLICE)], stage_v)
    pltpu.sync_copy(stage_v, agg_hbm.at[c, pl.ds(s * SLICE, SLICE)])


@functools.cache
def _sc_aggregate_call():
    mesh = plsc.VectorSubcoreMesh(core_axis_name="c", subcore_axis_name="s",
                                  num_cores=NC, num_subcores=NS)
    return pl.kernel(
        _sc_aggregate,
        out_type=jax.ShapeDtypeStruct((NC, NPAD, F), jnp.float32),
        mesh=mesh,
        scratch_types=[
            pltpu.VMEM((CHUNK,), jnp.int32),
            pltpu.VMEM((CHUNK,), jnp.int32),
            pltpu.VMEM((CHUNK, F), jnp.float32),
            pltpu.VMEM((SLICE, F), jnp.float32),
            pltpu.SemaphoreType.DMA,
            pltpu.VMEM_SHARED((NPAD, F), jnp.float32),
        ],
        compiler_params=pltpu.CompilerParams(use_tc_tiling_on_sc=False),
    )

BM = 1280


def _tc_y_body(x_ref, w_ref, deg_ref, y_ref):
    dis = lax.rsqrt(deg_ref[...])
    y_ref[...] = jnp.dot(x_ref[...], w_ref[...],
                         preferred_element_type=jnp.float32) * dis


_tc_y_call = pl.pallas_call(
    _tc_y_body,
    grid=(NPAD // BM,),
    in_specs=[
        pl.BlockSpec((BM, DIN), lambda m: (m, 0)),
        pl.BlockSpec((DIN, F), lambda m: (0, 0)),
        pl.BlockSpec((BM, 1), lambda m: (m, 0)),
    ],
    out_specs=pl.BlockSpec((BM, F), lambda m: (m, 0)),
    out_shape=jax.ShapeDtypeStruct((NPAD, F), jnp.float32),
)


def _tc_heads_body(y_ref, agg_ref, deg_ref, bc_ref, lw1_ref, lb1_ref, lw2_ref,
                   lb2_ref, o1_ref, o2_ref):
    dis = lax.rsqrt(deg_ref[...])
    feat = dis * (agg_ref[0] + agg_ref[1] + y_ref[...]) + bc_ref[...]
    h1 = jnp.dot(feat, lw1_ref[...],
                 preferred_element_type=jnp.float32) + lb1_ref[...]
    h2 = jnp.dot(feat, lw2_ref[...],
                 preferred_element_type=jnp.float32) + lb2_ref[...]
    n1 = jnp.sqrt(jnp.sum(h1 * h1, axis=1, keepdims=True))
    n2 = jnp.sqrt(jnp.sum(h2 * h2, axis=1, keepdims=True))
    o1_ref[...] = h1 / jnp.maximum(n1, 1e-12)
    o2_ref[...] = h2 / jnp.maximum(n2, 1e-12)


_tc_heads_call = pl.pallas_call(
    _tc_heads_body,
    grid=(NPAD // BM,),
    in_specs=[
        pl.BlockSpec((BM, F), lambda m: (m, 0)),
        pl.BlockSpec((NC, BM, F), lambda m: (0, m, 0)),
        pl.BlockSpec((BM, 1), lambda m: (m, 0)),
        pl.BlockSpec((1, F), lambda m: (0, 0)),
        pl.BlockSpec((F, DOUT), lambda m: (0, 0)),
        pl.BlockSpec((1, DOUT), lambda m: (0, 0)),
        pl.BlockSpec((F, DOUT), lambda m: (0, 0)),
        pl.BlockSpec((1, DOUT), lambda m: (0, 0)),
    ],
    out_specs=[
        pl.BlockSpec((BM, DOUT), lambda m: (m, 0)),
        pl.BlockSpec((BM, DOUT), lambda m: (m, 0)),
    ],
    out_shape=[
        jax.ShapeDtypeStruct((NPAD, DOUT), jnp.float32),
        jax.ShapeDtypeStruct((NPAD, DOUT), jnp.float32),
    ],
)


def kernel(x, edge_index, W1, b1, W2, b2, LW1, Lb1, LW2, Lb2):
    f32 = jnp.float32
    pad = jnp.full((EPAD - E,), N, jnp.int32)
    src_p = jnp.concatenate([edge_index[0], pad])
    dst_p = jnp.concatenate([edge_index[1], pad])
    xp = jnp.pad(x, ((0, NPAD - N), (0, 0)))
    Wc = jnp.zeros((DIN, F), f32).at[:, 0:DH].set(W1).at[:, 24:24 + DH].set(W2)
    bc = jnp.zeros((1, F), f32).at[0, 0:DH].set(b1).at[0, 24:24 + DH].set(b2)
    LW1p = jnp.zeros((F, DOUT), f32).at[0:DH].set(LW1)
    LW2p = jnp.zeros((F, DOUT), f32).at[24:24 + DH].set(LW2)
    ones = jnp.ones((CHUNK,), f32)
    zeros1 = jnp.zeros((NPAD,), f32)
    zeros2 = jnp.zeros((NPAD, F), f32)

    cnt = _sc_degree_call()(dst_p, ones, zeros1)
    deg = (1.0 + cnt[0] + cnt[1])[:, None]
    y = _tc_y_call(xp, Wc, deg)
    agg = _sc_aggregate_call()(src_p, dst_p, y, zeros2)
    o1, o2 = _tc_heads_call(y, agg, deg, bc, LW1p, Lb1[None], LW2p, Lb2[None])
    return (o1[:N], o2[:N])


# speedup vs baseline: 31.0474x; 1.2210x over previous
import functools

import jax
import jax.numpy as jnp
from jax import lax
from jax.experimental import pallas as pl
from jax.experimental.pallas import tpu as pltpu
from jax.experimental.pallas import tpu_sc as plsc

N = 10000
NPAD = 10240
E = 320000
DIN = 128
DH = 20
DOUT = 128
F = 48
NC, NS = 2, 16
NTILES = NC * NS
CHUNK = 128
EPT = 10240
EPAD = NTILES * EPT
NCHUNK = EPT // CHUNK
NPAIR = NCHUNK // 2
SLICE = NPAD // NS


def _sc_degree(dst_hbm, ones_hbm, zeros_hbm, cnt_hbm, idx0, idx1, ones_v,
               stage_v, si0, si1, ss0, ss1, cnt_sh):
    c = lax.axis_index("c")
    s = lax.axis_index("s")
    tid = c * NS + s
    ebase = tid * EPT
    idx = (idx0, idx1)
    si = (si0, si1)
    ss = (ss0, ss1)
    pltpu.sync_copy(zeros_hbm.at[pl.ds(s * SLICE, SLICE)], stage_v)
    pltpu.sync_copy(stage_v, cnt_sh.at[pl.ds(s * SLICE, SLICE)])
    pltpu.sync_copy(ones_hbm, ones_v)
    plsc.subcore_barrier()

    for b in (0, 1):
        pltpu.async_copy(dst_hbm.at[pl.ds(ebase + b * CHUNK, CHUNK)], idx[b],
                         si[b])

    def chunk(i, b, prefetch):
        pltpu.make_async_copy(dst_hbm.at[pl.ds(0, CHUNK)], idx[b],
                              si[b]).wait()
        sc = pltpu.async_copy(ones_v, cnt_sh.at[idx[b]], ss[b], add=True)
        sc.wait()
        if prefetch:
            pltpu.async_copy(
                dst_hbm.at[pl.ds(ebase + (i + 2) * CHUNK, CHUNK)], idx[b],
                si[b])

    def body(p, carry):
        for b in (0, 1):
            chunk(2 * p + b, b, True)
        return carry

    lax.fori_loop(0, NPAIR - 1, body, 0)
    for b in (0, 1):
        chunk(NCHUNK - 2 + b, b, False)
    plsc.subcore_barrier()
    pltpu.sync_copy(cnt_sh.at[pl.ds(s * SLICE, SLICE)], stage_v)
    pltpu.sync_copy(stage_v, cnt_hbm.at[c, pl.ds(s * SLICE, SLICE)])


@functools.cache
def _sc_degree_call():
    mesh = plsc.VectorSubcoreMesh(core_axis_name="c", subcore_axis_name="s",
                                  num_cores=NC, num_subcores=NS)
    return pl.kernel(
        _sc_degree,
        out_type=jax.ShapeDtypeStruct((NC, NPAD), jnp.float32),
        mesh=mesh,
        scratch_types=[
            pltpu.VMEM((CHUNK,), jnp.int32),
            pltpu.VMEM((CHUNK,), jnp.int32),
            pltpu.VMEM((CHUNK,), jnp.float32),
            pltpu.VMEM((SLICE,), jnp.float32),
            pltpu.SemaphoreType.DMA,
            pltpu.SemaphoreType.DMA,
            pltpu.SemaphoreType.DMA,
            pltpu.SemaphoreType.DMA,
            pltpu.VMEM_SHARED((NPAD,), jnp.float32),
        ],
        compiler_params=pltpu.CompilerParams(use_tc_tiling_on_sc=False),
    )


def _sc_aggregate(src_hbm, dst_hbm, y_hbm, zeros_hbm, agg_hbm, isrc0, isrc1,
                  idst0, idst1, rows0, rows1, stage_v, sis0, sis1, sid0, sid1,
                  sg0, sg1, ss0, ss1, agg_sh):
    c = lax.axis_index("c")
    s = lax.axis_index("s")
    tid = c * NS + s
    ebase = tid * EPT
    isrc = (isrc0, isrc1)
    idst = (idst0, idst1)
    rows = (rows0, rows1)
    sis = (sis0, sis1)
    sid = (sid0, sid1)
    sg = (sg0, sg1)
    ss = (ss0, ss1)
    pltpu.sync_copy(zeros_hbm.at[pl.ds(s * SLICE, SLICE)], stage_v)
    pltpu.sync_copy(stage_v, agg_sh.at[pl.ds(s * SLICE, SLICE)])
    plsc.subcore_barrier()

    def fetch(j, b):
        pltpu.async_copy(src_hbm.at[pl.ds(ebase + j * CHUNK, CHUNK)], isrc[b],
                         sis[b])
        pltpu.async_copy(dst_hbm.at[pl.ds(ebase + j * CHUNK, CHUNK)], idst[b],
                         sid[b])
        pltpu.make_async_copy(src_hbm.at[pl.ds(0, CHUNK)], isrc[b],
                              sis[b]).wait()
        pltpu.async_copy(y_hbm.at[isrc[b]], rows[b], sg[b])

    for b in (0, 1):
        fetch(b, b)

    def chunk(i, b, prefetch):
        pltpu.make_async_copy(y_hbm.at[pl.ds(0, CHUNK)], rows[b],
                              sg[b]).wait()
        pltpu.make_async_copy(dst_hbm.at[pl.ds(0, CHUNK)], idst[b],
                              sid[b]).wait()
        sc = pltpu.async_copy(rows[b], agg_sh.at[idst[b]], ss[b], add=True)
        sc.wait()
        if prefetch:
            fetch(i + 2, b)

    def body(p, carry):
        for b in (0, 1):
            chunk(2 * p + b, b, True)
        return carry

    lax.fori_loop(0, NPAIR - 1, body, 0)
    for b in (0, 1):
        chunk(NCHUNK - 2 + b, b, False)
    plsc.subcore_barrier()
    pltpu.sync_copy(agg_sh.at[pl.ds(s * SLICE, SLICE)], stage_v)
    pltpu.sync_copy(stage_v, agg_hbm.at[c, pl.ds(s * SLICE, SLICE)])


@functools.cache
def _sc_aggregate_call():
    mesh = plsc.VectorSubcoreMesh(core_axis_name="c", subcore_axis_name="s",
                                  num_cores=NC, num_subcores=NS)
    return pl.kernel(
        _sc_aggregate,
        out_type=jax.ShapeDtypeStruct((NC, NPAD, F), jnp.float32),
        mesh=mesh,
        scratch_types=[
            pltpu.VMEM((CHUNK,), jnp.int32),
            pltpu.VMEM((CHUNK,), jnp.int32),
            pltpu.VMEM((CHUNK,), jnp.int32),
            pltpu.VMEM((CHUNK,), jnp.int32),
            pltpu.VMEM((CHUNK, F), jnp.float32),
            pltpu.VMEM((CHUNK, F), jnp.float32),
            pltpu.VMEM((SLICE, F), jnp.float32),
            pltpu.SemaphoreType.DMA,
            pltpu.SemaphoreType.DMA,
            pltpu.SemaphoreType.DMA,
            pltpu.SemaphoreType.DMA,
            pltpu.SemaphoreType.DMA,
            pltpu.SemaphoreType.DMA,
            pltpu.SemaphoreType.DMA,
            pltpu.SemaphoreType.DMA,
            pltpu.VMEM_SHARED((NPAD, F), jnp.float32),
        ],
        compiler_params=pltpu.CompilerParams(use_tc_tiling_on_sc=False),
    )

BM = 1280


def _tc_y_body(x_ref, w_ref, deg_ref, y_ref):
    dis = lax.rsqrt(deg_ref[...])
    y_ref[...] = jnp.dot(x_ref[...], w_ref[...],
                         preferred_element_type=jnp.float32) * dis


_tc_y_call = pl.pallas_call(
    _tc_y_body,
    grid=(NPAD // BM,),
    in_specs=[
        pl.BlockSpec((BM, DIN), lambda m: (m, 0)),
        pl.BlockSpec((DIN, F), lambda m: (0, 0)),
        pl.BlockSpec((BM, 1), lambda m: (m, 0)),
    ],
    out_specs=pl.BlockSpec((BM, F), lambda m: (m, 0)),
    out_shape=jax.ShapeDtypeStruct((NPAD, F), jnp.float32),
)


def _tc_heads_body(y_ref, agg_ref, deg_ref, bc_ref, lw1_ref, lb1_ref, lw2_ref,
                   lb2_ref, o1_ref, o2_ref):
    dis = lax.rsqrt(deg_ref[...])
    feat = dis * (agg_ref[0] + agg_ref[1] + y_ref[...]) + bc_ref[...]
    h1 = jnp.dot(feat, lw1_ref[...],
                 preferred_element_type=jnp.float32) + lb1_ref[...]
    h2 = jnp.dot(feat, lw2_ref[...],
                 preferred_element_type=jnp.float32) + lb2_ref[...]
    n1 = jnp.sqrt(jnp.sum(h1 * h1, axis=1, keepdims=True))
    n2 = jnp.sqrt(jnp.sum(h2 * h2, axis=1, keepdims=True))
    o1_ref[...] = h1 / jnp.maximum(n1, 1e-12)
    o2_ref[...] = h2 / jnp.maximum(n2, 1e-12)


_tc_heads_call = pl.pallas_call(
    _tc_heads_body,
    grid=(NPAD // BM,),
    in_specs=[
        pl.BlockSpec((BM, F), lambda m: (m, 0)),
        pl.BlockSpec((NC, BM, F), lambda m: (0, m, 0)),
        pl.BlockSpec((BM, 1), lambda m: (m, 0)),
        pl.BlockSpec((1, F), lambda m: (0, 0)),
        pl.BlockSpec((F, DOUT), lambda m: (0, 0)),
        pl.BlockSpec((1, DOUT), lambda m: (0, 0)),
        pl.BlockSpec((F, DOUT), lambda m: (0, 0)),
        pl.BlockSpec((1, DOUT), lambda m: (0, 0)),
    ],
    out_specs=[
        pl.BlockSpec((BM, DOUT), lambda m: (m, 0)),
        pl.BlockSpec((BM, DOUT), lambda m: (m, 0)),
    ],
    out_shape=[
        jax.ShapeDtypeStruct((NPAD, DOUT), jnp.float32),
        jax.ShapeDtypeStruct((NPAD, DOUT), jnp.float32),
    ],
)


def kernel(x, edge_index, W1, b1, W2, b2, LW1, Lb1, LW2, Lb2):
    f32 = jnp.float32
    pad = jnp.full((EPAD - E,), N, jnp.int32)
    src_p = jnp.concatenate([edge_index[0], pad])
    dst_p = jnp.concatenate([edge_index[1], pad])
    xp = jnp.pad(x, ((0, NPAD - N), (0, 0)))
    Wc = jnp.zeros((DIN, F), f32).at[:, 0:DH].set(W1).at[:, 24:24 + DH].set(W2)
    bc = jnp.zeros((1, F), f32).at[0, 0:DH].set(b1).at[0, 24:24 + DH].set(b2)
    LW1p = jnp.zeros((F, DOUT), f32).at[0:DH].set(LW1)
    LW2p = jnp.zeros((F, DOUT), f32).at[24:24 + DH].set(LW2)
    ones = jnp.ones((CHUNK,), f32)
    zeros1 = jnp.zeros((NPAD,), f32)
    zeros2 = jnp.zeros((NPAD, F), f32)

    cnt = _sc_degree_call()(dst_p, ones, zeros1)
    deg = (1.0 + cnt[0] + cnt[1])[:, None]
    y = _tc_y_call(xp, Wc, deg)
    agg = _sc_aggregate_call()(src_p, dst_p, y, zeros2)
    o1, o2 = _tc_heads_call(y, agg, deg, bc, LW1p, Lb1[None], LW2p, Lb2[None])
    return (o1[:N], o2[:N])


# speedup vs baseline: 45.3849x; 1.4618x over previous
import functools

import jax
import jax.numpy as jnp
from jax import lax
from jax.experimental import pallas as pl
from jax.experimental.pallas import tpu as pltpu
from jax.experimental.pallas import tpu_sc as plsc

N = 10000
NPAD = 10240
E = 320000
DIN = 128
DH = 20
DOUT = 128
F = 48
NC, NS = 2, 16
NTILES = NC * NS
CHUNK = 128
EPT = 10240
EPAD = NTILES * EPT
NCHUNK = EPT // CHUNK
NPAIR = NCHUNK // 2
SLICE = NPAD // NS


def _sc_degree(dst_hbm, ones_hbm, zeros_hbm, cnt_hbm, idx0, idx1, ones_v,
               stage_v, si0, si1, ss0, ss1, cnt_sh):
    c = lax.axis_index("c")
    s = lax.axis_index("s")
    tid = c * NS + s
    ebase = tid * EPT
    idx = (idx0, idx1)
    si = (si0, si1)
    ss = (ss0, ss1)
    pltpu.sync_copy(zeros_hbm.at[pl.ds(s * SLICE, SLICE)], stage_v)
    pltpu.sync_copy(stage_v, cnt_sh.at[pl.ds(s * SLICE, SLICE)])
    pltpu.sync_copy(ones_hbm, ones_v)
    plsc.subcore_barrier()

    for b in (0, 1):
        pltpu.async_copy(dst_hbm.at[pl.ds(ebase + b * CHUNK, CHUNK)], idx[b],
                         si[b])

    def chunk(i, b, prefetch):
        pltpu.make_async_copy(dst_hbm.at[pl.ds(0, CHUNK)], idx[b],
                              si[b]).wait()
        sc = pltpu.async_copy(ones_v, cnt_sh.at[idx[b]], ss[b], add=True)
        sc.wait()
        if prefetch:
            pltpu.async_copy(
                dst_hbm.at[pl.ds(ebase + (i + 2) * CHUNK, CHUNK)], idx[b],
                si[b])

    def body(p, carry):
        for b in (0, 1):
            chunk(2 * p + b, b, True)
        return carry

    lax.fori_loop(0, NPAIR - 1, body, 0)
    for b in (0, 1):
        chunk(NCHUNK - 2 + b, b, False)
    plsc.subcore_barrier()
    pltpu.sync_copy(cnt_sh.at[pl.ds(s * SLICE, SLICE)], stage_v)
    pltpu.sync_copy(stage_v, cnt_hbm.at[c, pl.ds(s * SLICE, SLICE)])


@functools.cache
def _sc_degree_call():
    mesh = plsc.VectorSubcoreMesh(core_axis_name="c", subcore_axis_name="s",
                                  num_cores=NC, num_subcores=NS)
    return pl.kernel(
        _sc_degree,
        out_type=jax.ShapeDtypeStruct((NC, NPAD), jnp.float32),
        mesh=mesh,
        scratch_types=[
            pltpu.VMEM((CHUNK,), jnp.int32),
            pltpu.VMEM((CHUNK,), jnp.int32),
            pltpu.VMEM((CHUNK,), jnp.float32),
            pltpu.VMEM((SLICE,), jnp.float32),
            pltpu.SemaphoreType.DMA,
            pltpu.SemaphoreType.DMA,
            pltpu.SemaphoreType.DMA,
            pltpu.SemaphoreType.DMA,
            pltpu.VMEM_SHARED((NPAD,), jnp.float32),
        ],
        compiler_params=pltpu.CompilerParams(use_tc_tiling_on_sc=False),
    )


def _sc_aggregate(src_hbm, dst_hbm, y_hbm, zeros_hbm, agg_hbm, isrc0, isrc1,
                  idst0, idst1, rows0, rows1, stage_v, sis0, sis1, sid0, sid1,
                  sg0, sg1, ss0, ss1, agg_sh, y_sh):
    c = lax.axis_index("c")
    s = lax.axis_index("s")
    tid = c * NS + s
    ebase = tid * EPT
    isrc = (isrc0, isrc1)
    idst = (idst0, idst1)
    rows = (rows0, rows1)
    sis = (sis0, sis1)
    sid = (sid0, sid1)
    sg = (sg0, sg1)
    ss = (ss0, ss1)
    pltpu.sync_copy(zeros_hbm.at[pl.ds(s * SLICE, SLICE)], stage_v)
    pltpu.sync_copy(stage_v, agg_sh.at[pl.ds(s * SLICE, SLICE)])
    pltpu.sync_copy(y_hbm.at[pl.ds(s * SLICE, SLICE)], stage_v)
    pltpu.sync_copy(stage_v, y_sh.at[pl.ds(s * SLICE, SLICE)])
    plsc.subcore_barrier()

    def fetch(j, b):
        pltpu.async_copy(src_hbm.at[pl.ds(ebase + j * CHUNK, CHUNK)], isrc[b],
                         sis[b])
        pltpu.async_copy(dst_hbm.at[pl.ds(ebase + j * CHUNK, CHUNK)], idst[b],
                         sid[b])
        pltpu.make_async_copy(src_hbm.at[pl.ds(0, CHUNK)], isrc[b],
                              sis[b]).wait()
        pltpu.async_copy(y_sh.at[isrc[b]], rows[b], sg[b])

    for b in (0, 1):
        fetch(b, b)

    def chunk(i, b, prefetch):
        pltpu.make_async_copy(y_sh.at[pl.ds(0, CHUNK)], rows[b],
                              sg[b]).wait()
        pltpu.make_async_copy(dst_hbm.at[pl.ds(0, CHUNK)], idst[b],
                              sid[b]).wait()
        sc = pltpu.async_copy(rows[b], agg_sh.at[idst[b]], ss[b], add=True)
        sc.wait()
        if prefetch:
            fetch(i + 2, b)

    def body(p, carry):
        for b in (0, 1):
            chunk(2 * p + b, b, True)
        return carry

    lax.fori_loop(0, NPAIR - 1, body, 0)
    for b in (0, 1):
        chunk(NCHUNK - 2 + b, b, False)
    plsc.subcore_barrier()
    pltpu.sync_copy(agg_sh.at[pl.ds(s * SLICE, SLICE)], stage_v)
    pltpu.sync_copy(stage_v, agg_hbm.at[c, pl.ds(s * SLICE, SLICE)])


@functools.cache
def _sc_aggregate_call():
    mesh = plsc.VectorSubcoreMesh(core_axis_name="c", subcore_axis_name="s",
                                  num_cores=NC, num_subcores=NS)
    return pl.kernel(
        _sc_aggregate,
        out_type=jax.ShapeDtypeStruct((NC, NPAD, F), jnp.float32),
        mesh=mesh,
        scratch_types=[
            pltpu.VMEM((CHUNK,), jnp.int32),
            pltpu.VMEM((CHUNK,), jnp.int32),
            pltpu.VMEM((CHUNK,), jnp.int32),
            pltpu.VMEM((CHUNK,), jnp.int32),
            pltpu.VMEM((CHUNK, F), jnp.float32),
            pltpu.VMEM((CHUNK, F), jnp.float32),
            pltpu.VMEM((SLICE, F), jnp.float32),
            pltpu.SemaphoreType.DMA,
            pltpu.SemaphoreType.DMA,
            pltpu.SemaphoreType.DMA,
            pltpu.SemaphoreType.DMA,
            pltpu.SemaphoreType.DMA,
            pltpu.SemaphoreType.DMA,
            pltpu.SemaphoreType.DMA,
            pltpu.SemaphoreType.DMA,
            pltpu.VMEM_SHARED((NPAD, F), jnp.float32),
            pltpu.VMEM_SHARED((NPAD, F), jnp.float32),
        ],
        compiler_params=pltpu.CompilerParams(use_tc_tiling_on_sc=False),
    )

BM = 1280


def _tc_y_body(x_ref, w_ref, deg_ref, y_ref):
    dis = lax.rsqrt(deg_ref[...])
    y_ref[...] = jnp.dot(x_ref[...], w_ref[...],
                         preferred_element_type=jnp.float32) * dis


_tc_y_call = pl.pallas_call(
    _tc_y_body,
    grid=(NPAD // BM,),
    in_specs=[
        pl.BlockSpec((BM, DIN), lambda m: (m, 0)),
        pl.BlockSpec((DIN, F), lambda m: (0, 0)),
        pl.BlockSpec((BM, 1), lambda m: (m, 0)),
    ],
    out_specs=pl.BlockSpec((BM, F), lambda m: (m, 0)),
    out_shape=jax.ShapeDtypeStruct((NPAD, F), jnp.float32),
)


def _tc_heads_body(y_ref, agg_ref, deg_ref, bc_ref, lw1_ref, lb1_ref, lw2_ref,
                   lb2_ref, o1_ref, o2_ref):
    dis = lax.rsqrt(deg_ref[...])
    feat = dis * (agg_ref[0] + agg_ref[1] + y_ref[...]) + bc_ref[...]
    h1 = jnp.dot(feat, lw1_ref[...],
                 preferred_element_type=jnp.float32) + lb1_ref[...]
    h2 = jnp.dot(feat, lw2_ref[...],
                 preferred_element_type=jnp.float32) + lb2_ref[...]
    n1 = jnp.sqrt(jnp.sum(h1 * h1, axis=1, keepdims=True))
    n2 = jnp.sqrt(jnp.sum(h2 * h2, axis=1, keepdims=True))
    o1_ref[...] = h1 / jnp.maximum(n1, 1e-12)
    o2_ref[...] = h2 / jnp.maximum(n2, 1e-12)


_tc_heads_call = pl.pallas_call(
    _tc_heads_body,
    grid=(NPAD // BM,),
    in_specs=[
        pl.BlockSpec((BM, F), lambda m: (m, 0)),
        pl.BlockSpec((NC, BM, F), lambda m: (0, m, 0)),
        pl.BlockSpec((BM, 1), lambda m: (m, 0)),
        pl.BlockSpec((1, F), lambda m: (0, 0)),
        pl.BlockSpec((F, DOUT), lambda m: (0, 0)),
        pl.BlockSpec((1, DOUT), lambda m: (0, 0)),
        pl.BlockSpec((F, DOUT), lambda m: (0, 0)),
        pl.BlockSpec((1, DOUT), lambda m: (0, 0)),
    ],
    out_specs=[
        pl.BlockSpec((BM, DOUT), lambda m: (m, 0)),
        pl.BlockSpec((BM, DOUT), lambda m: (m, 0)),
    ],
    out_shape=[
        jax.ShapeDtypeStruct((NPAD, DOUT), jnp.float32),
        jax.ShapeDtypeStruct((NPAD, DOUT), jnp.float32),
    ],
)


def kernel(x, edge_index, W1, b1, W2, b2, LW1, Lb1, LW2, Lb2):
    f32 = jnp.float32
    pad = jnp.full((EPAD - E,), N, jnp.int32)
    src_p = jnp.concatenate([edge_index[0], pad])
    dst_p = jnp.concatenate([edge_index[1], pad])
    xp = jnp.pad(x, ((0, NPAD - N), (0, 0)))
    Wc = jnp.zeros((DIN, F), f32).at[:, 0:DH].set(W1).at[:, 24:24 + DH].set(W2)
    bc = jnp.zeros((1, F), f32).at[0, 0:DH].set(b1).at[0, 24:24 + DH].set(b2)
    LW1p = jnp.zeros((F, DOUT), f32).at[0:DH].set(LW1)
    LW2p = jnp.zeros((F, DOUT), f32).at[24:24 + DH].set(LW2)
    ones = jnp.ones((CHUNK,), f32)
    zeros1 = jnp.zeros((NPAD,), f32)
    zeros2 = jnp.zeros((NPAD, F), f32)

    cnt = _sc_degree_call()(dst_p, ones, zeros1)
    deg = (1.0 + cnt[0] + cnt[1])[:, None]
    y = _tc_y_call(xp, Wc, deg)
    agg = _sc_aggregate_call()(src_p, dst_p, y, zeros2)
    o1, o2 = _tc_heads_call(y, agg, deg, bc, LW1p, Lb1[None], LW2p, Lb2[None])
    return (o1[:N], o2[:N])


# speedup vs baseline: 47.3168x; 1.0426x over previous
import functools

import jax
import jax.numpy as jnp
from jax import lax
from jax.experimental import pallas as pl
from jax.experimental.pallas import tpu as pltpu
from jax.experimental.pallas import tpu_sc as plsc

N = 10000
NPAD = 10240
E = 320000
DIN = 128
DH = 20
DOUT = 128
F = 40
NC, NS = 2, 16
NTILES = NC * NS
CHUNK = 128
EPT = 10240
EPAD = NTILES * EPT
NCHUNK = EPT // CHUNK
NPAIR = NCHUNK // 2
SLICE = NPAD // NS


def _sc_degree(dst_hbm, ones_hbm, zeros_hbm, cnt_hbm, idx0, idx1, ones_v,
               stage_v, si0, si1, ss0, ss1, cnt_sh):
    c = lax.axis_index("c")
    s = lax.axis_index("s")
    tid = c * NS + s
    ebase = tid * EPT
    idx = (idx0, idx1)
    si = (si0, si1)
    ss = (ss0, ss1)
    pltpu.sync_copy(zeros_hbm.at[pl.ds(s * SLICE, SLICE)], stage_v)
    pltpu.sync_copy(stage_v, cnt_sh.at[pl.ds(s * SLICE, SLICE)])
    pltpu.sync_copy(ones_hbm, ones_v)
    plsc.subcore_barrier()

    for b in (0, 1):
        pltpu.async_copy(dst_hbm.at[pl.ds(ebase + b * CHUNK, CHUNK)], idx[b],
                         si[b])

    def chunk(i, b, prefetch):
        pltpu.make_async_copy(dst_hbm.at[pl.ds(0, CHUNK)], idx[b],
                              si[b]).wait()
        sc = pltpu.async_copy(ones_v, cnt_sh.at[idx[b]], ss[b], add=True)
        sc.wait()
        if prefetch:
            pltpu.async_copy(
                dst_hbm.at[pl.ds(ebase + (i + 2) * CHUNK, CHUNK)], idx[b],
                si[b])

    def body(p, carry):
        for b in (0, 1):
            chunk(2 * p + b, b, True)
        return carry

    lax.fori_loop(0, NPAIR - 1, body, 0)
    for b in (0, 1):
        chunk(NCHUNK - 2 + b, b, False)
    plsc.subcore_barrier()
    pltpu.sync_copy(cnt_sh.at[pl.ds(s * SLICE, SLICE)], stage_v)
    pltpu.sync_copy(stage_v, cnt_hbm.at[c, pl.ds(s * SLICE, SLICE)])


@functools.cache
def _sc_degree_call():
    mesh = plsc.VectorSubcoreMesh(core_axis_name="c", subcore_axis_name="s",
                                  num_cores=NC, num_subcores=NS)
    return pl.kernel(
        _sc_degree,
        out_type=jax.ShapeDtypeStruct((NC, NPAD), jnp.float32),
        mesh=mesh,
        scratch_types=[
            pltpu.VMEM((CHUNK,), jnp.int32),
            pltpu.VMEM((CHUNK,), jnp.int32),
            pltpu.VMEM((CHUNK,), jnp.float32),
            pltpu.VMEM((SLICE,), jnp.float32),
            pltpu.SemaphoreType.DMA,
            pltpu.SemaphoreType.DMA,
            pltpu.SemaphoreType.DMA,
            pltpu.SemaphoreType.DMA,
            pltpu.VMEM_SHARED((NPAD,), jnp.float32),
        ],
        compiler_params=pltpu.CompilerParams(use_tc_tiling_on_sc=False),
    )


def _sc_aggregate(src_hbm, dst_hbm, y_hbm, zeros_hbm, agg_hbm, isrc0, isrc1,
                  idst0, idst1, rows0, rows1, stage_v, sis0, sis1, sid0, sid1,
                  sg0, sg1, ss0, ss1, agg_sh, y_sh):
    c = lax.axis_index("c")
    s = lax.axis_index("s")
    tid = c * NS + s
    ebase = tid * EPT
    isrc = (isrc0, isrc1)
    idst = (idst0, idst1)
    rows = (rows0, rows1)
    sis = (sis0, sis1)
    sid = (sid0, sid1)
    sg = (sg0, sg1)
    ss = (ss0, ss1)
    pltpu.sync_copy(zeros_hbm.at[pl.ds(s * SLICE, SLICE)], stage_v)
    pltpu.sync_copy(stage_v, agg_sh.at[pl.ds(s * SLICE, SLICE)])
    pltpu.sync_copy(y_hbm.at[pl.ds(s * SLICE, SLICE)], stage_v)
    pltpu.sync_copy(stage_v, y_sh.at[pl.ds(s * SLICE, SLICE)])
    plsc.subcore_barrier()

    def fetch(j, b):
        pltpu.async_copy(src_hbm.at[pl.ds(ebase + j * CHUNK, CHUNK)], isrc[b],
                         sis[b])
        pltpu.async_copy(dst_hbm.at[pl.ds(ebase + j * CHUNK, CHUNK)], idst[b],
                         sid[b])
        pltpu.make_async_copy(src_hbm.at[pl.ds(0, CHUNK)], isrc[b],
                              sis[b]).wait()
        pltpu.async_copy(y_sh.at[isrc[b]], rows[b], sg[b])

    for b in (0, 1):
        fetch(b, b)

    def chunk(i, b, prefetch):
        pltpu.make_async_copy(y_sh.at[pl.ds(0, CHUNK)], rows[b],
                              sg[b]).wait()
        pltpu.make_async_copy(dst_hbm.at[pl.ds(0, CHUNK)], idst[b],
                              sid[b]).wait()
        sc = pltpu.async_copy(rows[b], agg_sh.at[idst[b]], ss[b], add=True)
        sc.wait()
        if prefetch:
            fetch(i + 2, b)

    def body(p, carry):
        for b in (0, 1):
            chunk(2 * p + b, b, True)
        return carry

    lax.fori_loop(0, NPAIR - 1, body, 0)
    for b in (0, 1):
        chunk(NCHUNK - 2 + b, b, False)
    plsc.subcore_barrier()
    pltpu.sync_copy(agg_sh.at[pl.ds(s * SLICE, SLICE)], stage_v)
    pltpu.sync_copy(stage_v, agg_hbm.at[c, pl.ds(s * SLICE, SLICE)])


@functools.cache
def _sc_aggregate_call():
    mesh = plsc.VectorSubcoreMesh(core_axis_name="c", subcore_axis_name="s",
                                  num_cores=NC, num_subcores=NS)
    return pl.kernel(
        _sc_aggregate,
        out_type=jax.ShapeDtypeStruct((NC, NPAD, F), jnp.float32),
        mesh=mesh,
        scratch_types=[
            pltpu.VMEM((CHUNK,), jnp.int32),
            pltpu.VMEM((CHUNK,), jnp.int32),
            pltpu.VMEM((CHUNK,), jnp.int32),
            pltpu.VMEM((CHUNK,), jnp.int32),
            pltpu.VMEM((CHUNK, F), jnp.float32),
            pltpu.VMEM((CHUNK, F), jnp.float32),
            pltpu.VMEM((SLICE, F), jnp.float32),
            pltpu.SemaphoreType.DMA,
            pltpu.SemaphoreType.DMA,
            pltpu.SemaphoreType.DMA,
            pltpu.SemaphoreType.DMA,
            pltpu.SemaphoreType.DMA,
            pltpu.SemaphoreType.DMA,
            pltpu.SemaphoreType.DMA,
            pltpu.SemaphoreType.DMA,
            pltpu.VMEM_SHARED((NPAD, F), jnp.float32),
            pltpu.VMEM_SHARED((NPAD, F), jnp.float32),
        ],
        compiler_params=pltpu.CompilerParams(use_tc_tiling_on_sc=False),
    )

BM = 1280


def _tc_xw_body(x_ref, w_ref, xw_ref):
    xw_ref[...] = jnp.dot(x_ref[...], w_ref[...],
                          preferred_element_type=jnp.float32)


_tc_xw_call = pl.pallas_call(
    _tc_xw_body,
    grid=(NPAD // BM,),
    in_specs=[
        pl.BlockSpec((BM, DIN), lambda m: (m, 0)),
        pl.BlockSpec((DIN, F), lambda m: (0, 0)),
    ],
    out_specs=pl.BlockSpec((BM, F), lambda m: (m, 0)),
    out_shape=jax.ShapeDtypeStruct((NPAD, F), jnp.float32),
)


def _tc_scale_body(xw_ref, deg_ref, y_ref):
    y_ref[...] = xw_ref[...] * lax.rsqrt(deg_ref[...])


_tc_scale_call = pl.pallas_call(
    _tc_scale_body,
    grid=(NPAD // BM,),
    in_specs=[
        pl.BlockSpec((BM, F), lambda m: (m, 0)),
        pl.BlockSpec((BM, 1), lambda m: (m, 0)),
    ],
    out_specs=pl.BlockSpec((BM, F), lambda m: (m, 0)),
    out_shape=jax.ShapeDtypeStruct((NPAD, F), jnp.float32),
)


def _tc_heads_body(y_ref, agg_ref, deg_ref, bc_ref, lw1_ref, lb1_ref, lw2_ref,
                   lb2_ref, o1_ref, o2_ref):
    dis = lax.rsqrt(deg_ref[...])
    feat = dis * (agg_ref[0] + agg_ref[1] + y_ref[...]) + bc_ref[...]
    h1 = jnp.dot(feat, lw1_ref[...],
                 preferred_element_type=jnp.float32) + lb1_ref[...]
    h2 = jnp.dot(feat, lw2_ref[...],
                 preferred_element_type=jnp.float32) + lb2_ref[...]
    n1 = jnp.sqrt(jnp.sum(h1 * h1, axis=1, keepdims=True))
    n2 = jnp.sqrt(jnp.sum(h2 * h2, axis=1, keepdims=True))
    o1_ref[...] = h1 / jnp.maximum(n1, 1e-12)
    o2_ref[...] = h2 / jnp.maximum(n2, 1e-12)


_tc_heads_call = pl.pallas_call(
    _tc_heads_body,
    grid=(NPAD // BM,),
    in_specs=[
        pl.BlockSpec((BM, F), lambda m: (m, 0)),
        pl.BlockSpec((NC, BM, F), lambda m: (0, m, 0)),
        pl.BlockSpec((BM, 1), lambda m: (m, 0)),
        pl.BlockSpec((1, F), lambda m: (0, 0)),
        pl.BlockSpec((F, DOUT), lambda m: (0, 0)),
        pl.BlockSpec((1, DOUT), lambda m: (0, 0)),
        pl.BlockSpec((F, DOUT), lambda m: (0, 0)),
        pl.BlockSpec((1, DOUT), lambda m: (0, 0)),
    ],
    out_specs=[
        pl.BlockSpec((BM, DOUT), lambda m: (m, 0)),
        pl.BlockSpec((BM, DOUT), lambda m: (m, 0)),
    ],
    out_shape=[
        jax.ShapeDtypeStruct((NPAD, DOUT), jnp.float32),
        jax.ShapeDtypeStruct((NPAD, DOUT), jnp.float32),
    ],
)


def kernel(x, edge_index, W1, b1, W2, b2, LW1, Lb1, LW2, Lb2):
    f32 = jnp.float32
    pad = jnp.full((EPAD - E,), N, jnp.int32)
    src_p = jnp.concatenate([edge_index[0], pad])
    dst_p = jnp.concatenate([edge_index[1], pad])
    xp = jnp.pad(x, ((0, NPAD - N), (0, 0)))
    Wc = jnp.concatenate([W1, W2], axis=1)
    bc = jnp.concatenate([b1, b2])[None, :]
    LW1p = jnp.zeros((F, DOUT), f32).at[0:DH].set(LW1)
    LW2p = jnp.zeros((F, DOUT), f32).at[DH:2 * DH].set(LW2)
    ones = jnp.ones((CHUNK,), f32)
    zeros1 = jnp.zeros((NPAD,), f32)
    zeros2 = jnp.zeros((NPAD, F), f32)

    xw = _tc_xw_call(xp, Wc)
    cnt = _sc_degree_call()(dst_p, ones, zeros1)
    deg = (1.0 + cnt[0] + cnt[1])[:, None]
    y = _tc_scale_call(xw, deg)
    agg = _sc_aggregate_call()(src_p, dst_p, y, zeros2)
    o1, o2 = _tc_heads_call(y, agg, deg, bc, LW1p, Lb1[None], LW2p, Lb2[None])
    return (o1[:N], o2[:N])


# speedup vs baseline: 57.4740x; 1.2147x over previous
import functools

import jax
import jax.numpy as jnp
from jax import lax
from jax.experimental import pallas as pl
from jax.experimental.pallas import tpu as pltpu
from jax.experimental.pallas import tpu_sc as plsc

N = 10000
NPAD = 10240
E = 320000
DIN = 128
DH = 20
DOUT = 128
F = 40
NC, NS = 2, 16
NTILES = NC * NS
CHUNK = 128
EPT = 10240
EPAD = NTILES * EPT
NCHUNK = EPT // CHUNK
NPAIR = NCHUNK // 2
SLICE = NPAD // NS


def _sc_degree(edge_hbm, ones_hbm, zeros_hbm, cnt_hbm, idx0, idx1, ones_v,
               stage_v, si0, si1, ss0, ss1, cnt_sh):
    c = lax.axis_index("c")
    s = lax.axis_index("s")
    tid = c * NS + s
    ebase = tid * EPT
    idx = (idx0, idx1)
    si = (si0, si1)
    ss = (ss0, ss1)
    pltpu.sync_copy(zeros_hbm.at[pl.ds(s * SLICE, SLICE)], stage_v)
    pltpu.sync_copy(stage_v, cnt_sh.at[pl.ds(s * SLICE, SLICE)])
    pltpu.sync_copy(ones_hbm, ones_v)
    plsc.subcore_barrier()

    for b in (0, 1):
        pltpu.async_copy(edge_hbm.at[1, pl.ds(ebase + b * CHUNK, CHUNK)],
                         idx[b], si[b])

    def chunk(i, b, prefetch):
        pltpu.make_async_copy(edge_hbm.at[1, pl.ds(0, CHUNK)], idx[b],
                              si[b]).wait()
        sc = pltpu.async_copy(ones_v, cnt_sh.at[idx[b]], ss[b], add=True)
        sc.wait()
        if prefetch:
            pltpu.async_copy(
                edge_hbm.at[1, pl.ds(ebase + (i + 2) * CHUNK, CHUNK)], idx[b],
                si[b])

    def body(p, carry):
        for b in (0, 1):
            chunk(2 * p + b, b, True)
        return carry

    lax.fori_loop(0, NPAIR - 1, body, 0)
    for b in (0, 1):
        chunk(NCHUNK - 2 + b, b, False)
    plsc.subcore_barrier()
    pltpu.sync_copy(cnt_sh.at[pl.ds(s * SLICE, SLICE)], stage_v)
    pltpu.sync_copy(stage_v, cnt_hbm.at[c, pl.ds(s * SLICE, SLICE)])


@functools.cache
def _sc_degree_call():
    mesh = plsc.VectorSubcoreMesh(core_axis_name="c", subcore_axis_name="s",
                                  num_cores=NC, num_subcores=NS)
    return pl.kernel(
        _sc_degree,
        out_type=jax.ShapeDtypeStruct((NC, NPAD), jnp.float32),
        mesh=mesh,
        scratch_types=[
            pltpu.VMEM((CHUNK,), jnp.int32),
            pltpu.VMEM((CHUNK,), jnp.int32),
            pltpu.VMEM((CHUNK,), jnp.float32),
            pltpu.VMEM((SLICE,), jnp.float32),
            pltpu.SemaphoreType.DMA,
            pltpu.SemaphoreType.DMA,
            pltpu.SemaphoreType.DMA,
            pltpu.SemaphoreType.DMA,
            pltpu.VMEM_SHARED((NPAD,), jnp.float32),
        ],
        compiler_params=pltpu.CompilerParams(use_tc_tiling_on_sc=False),
    )


def _sc_aggregate(edge_hbm, xw_hbm, cnt_hbm, zeros_hbm, agg_hbm, isrc0, isrc1,
                  idst0, idst1, rows0, rows1, stage_v, stage2_v, c0_v, c1_v,
                  dis_v, sis0, sis1, sid0, sid1, sg0, sg1, ss0, ss1, agg_sh,
                  y_sh):
    c = lax.axis_index("c")
    s = lax.axis_index("s")
    tid = c * NS + s
    ebase = tid * EPT
    isrc = (isrc0, isrc1)
    idst = (idst0, idst1)
    rows = (rows0, rows1)
    sis = (sis0, sis1)
    sid = (sid0, sid1)
    sg = (sg0, sg1)
    ss = (ss0, ss1)
    pltpu.sync_copy(zeros_hbm.at[pl.ds(s * SLICE, SLICE)], stage_v)
    pltpu.sync_copy(stage_v, agg_sh.at[pl.ds(s * SLICE, SLICE)])
    pltpu.sync_copy(xw_hbm.at[pl.ds(s * SLICE, SLICE)], stage_v)
    pltpu.sync_copy(cnt_hbm.at[0, pl.ds(s * SLICE, SLICE)], c0_v)
    pltpu.sync_copy(cnt_hbm.at[1, pl.ds(s * SLICE, SLICE)], c1_v)

    def disbody(k, carry):
        d = 1.0 + c0_v[pl.ds(16 * k, 16)] + c1_v[pl.ds(16 * k, 16)]
        g = plsc.bitcast(0x5F3759DF - (plsc.bitcast(d, jnp.int32) >> 1),
                         jnp.float32)
        g = g * (1.5 - 0.5 * d * g * g)
        g = g * (1.5 - 0.5 * d * g * g)
        g = g * (1.5 - 0.5 * d * g * g)
        dis_v[pl.ds(16 * k, 16)] = g
        return carry

    lax.fori_loop(0, SLICE // 16, disbody, 0)

    def rowblock(k, carry):
        dv = dis_v[pl.ds(16 * k, 16)]
        for j in range(16):
            r = 16 * k + j
            ga = dv[j]
            stage2_v[r, pl.ds(0, 16)] = stage_v[r, pl.ds(0, 16)] * ga
            stage2_v[r, pl.ds(16, 16)] = stage_v[r, pl.ds(16, 16)] * ga
            stage2_v[r, pl.ds(24, 16)] = stage_v[r, pl.ds(24, 16)] * ga
        return carry

    lax.fori_loop(0, SLICE // 16, rowblock, 0)
    pltpu.sync_copy(stage2_v, y_sh.at[pl.ds(s * SLICE, SLICE)])
    plsc.subcore_barrier()

    def fetch(j, b):
        pltpu.async_copy(edge_hbm.at[0, pl.ds(ebase + j * CHUNK, CHUNK)],
                         isrc[b], sis[b])
        pltpu.async_copy(edge_hbm.at[1, pl.ds(ebase + j * CHUNK, CHUNK)],
                         idst[b], sid[b])
        pltpu.make_async_copy(edge_hbm.at[0, pl.ds(0, CHUNK)], isrc[b],
                              sis[b]).wait()
        pltpu.async_copy(y_sh.at[isrc[b]], rows[b], sg[b])

    for b in (0, 1):
        fetch(b, b)

    def chunk(i, b, prefetch):
        pltpu.make_async_copy(y_sh.at[pl.ds(0, CHUNK)], rows[b],
                              sg[b]).wait()
        pltpu.make_async_copy(edge_hbm.at[1, pl.ds(0, CHUNK)], idst[b],
                              sid[b]).wait()
        sc = pltpu.async_copy(rows[b], agg_sh.at[idst[b]], ss[b], add=True)
        sc.wait()
        if prefetch:
            fetch(i + 2, b)

    def body(p, carry):
        for b in (0, 1):
            chunk(2 * p + b, b, True)
        return carry

    lax.fori_loop(0, NPAIR - 1, body, 0)
    for b in (0, 1):
        chunk(NCHUNK - 2 + b, b, False)
    plsc.subcore_barrier()
    pltpu.sync_copy(agg_sh.at[pl.ds(s * SLICE, SLICE)], stage_v)
    pltpu.sync_copy(stage_v, agg_hbm.at[c, pl.ds(s * SLICE, SLICE)])


@functools.cache
def _sc_aggregate_call():
    mesh = plsc.VectorSubcoreMesh(core_axis_name="c", subcore_axis_name="s",
                                  num_cores=NC, num_subcores=NS)
    return pl.kernel(
        _sc_aggregate,
        out_type=jax.ShapeDtypeStruct((NC, NPAD, F), jnp.float32),
        mesh=mesh,
        scratch_types=[
            pltpu.VMEM((CHUNK,), jnp.int32),
            pltpu.VMEM((CHUNK,), jnp.int32),
            pltpu.VMEM((CHUNK,), jnp.int32),
            pltpu.VMEM((CHUNK,), jnp.int32),
            pltpu.VMEM((CHUNK, F), jnp.float32),
            pltpu.VMEM((CHUNK, F), jnp.float32),
            pltpu.VMEM((SLICE, F), jnp.float32),
            pltpu.VMEM((SLICE, F), jnp.float32),
            pltpu.VMEM((SLICE,), jnp.float32),
            pltpu.VMEM((SLICE,), jnp.float32),
            pltpu.VMEM((SLICE,), jnp.float32),
            pltpu.SemaphoreType.DMA,
            pltpu.SemaphoreType.DMA,
            pltpu.SemaphoreType.DMA,
            pltpu.SemaphoreType.DMA,
            pltpu.SemaphoreType.DMA,
            pltpu.SemaphoreType.DMA,
            pltpu.SemaphoreType.DMA,
            pltpu.SemaphoreType.DMA,
            pltpu.VMEM_SHARED((NPAD, F), jnp.float32),
            pltpu.VMEM_SHARED((NPAD, F), jnp.float32),
        ],
        compiler_params=pltpu.CompilerParams(use_tc_tiling_on_sc=False,
                                             needs_layout_passes=False),
    )


BM = 2000


def _tc_xw_body(x_ref, w_ref, xw_ref):
    xw_ref[...] = jnp.dot(x_ref[...], w_ref[...],
                          preferred_element_type=jnp.float32)


_tc_xw_call = pl.pallas_call(
    _tc_xw_body,
    grid=(N // BM,),
    in_specs=[
        pl.BlockSpec((BM, DIN), lambda m: (m, 0)),
        pl.BlockSpec((DIN, F), lambda m: (0, 0)),
    ],
    out_specs=pl.BlockSpec((BM, F), lambda m: (m, 0)),
    out_shape=jax.ShapeDtypeStruct((NPAD, F), jnp.float32),
)


def _tc_heads_body(xw_ref, agg_ref, cntT_ref, bc_ref, lw1_ref, lb1_ref,
                   lw2_ref, lb2_ref, o1_ref, o2_ref):
    deg = 1.0 + cntT_ref[:, 0:1] + cntT_ref[:, 1:2]
    dis = lax.rsqrt(deg)
    feat = dis * (agg_ref[0] + agg_ref[1]) + (dis * dis) * xw_ref[...] \
        + bc_ref[...]
    h1 = jnp.dot(feat, lw1_ref[...],
                 preferred_element_type=jnp.float32) + lb1_ref[...]
    h2 = jnp.dot(feat, lw2_ref[...],
                 preferred_element_type=jnp.float32) + lb2_ref[...]
    n1 = jnp.sqrt(jnp.sum(h1 * h1, axis=1, keepdims=True))
    n2 = jnp.sqrt(jnp.sum(h2 * h2, axis=1, keepdims=True))
    o1_ref[...] = h1 / jnp.maximum(n1, 1e-12)
    o2_ref[...] = h2 / jnp.maximum(n2, 1e-12)


_tc_heads_call = pl.pallas_call(
    _tc_heads_body,
    grid=(N // BM,),
    in_specs=[
        pl.BlockSpec((BM, F), lambda m: (m, 0)),
        pl.BlockSpec((NC, BM, F), lambda m: (0, m, 0)),
        pl.BlockSpec((BM, NC), lambda m: (m, 0)),
        pl.BlockSpec((1, F), lambda m: (0, 0)),
        pl.BlockSpec((F, DOUT), lambda m: (0, 0)),
        pl.BlockSpec((1, DOUT), lambda m: (0, 0)),
        pl.BlockSpec((F, DOUT), lambda m: (0, 0)),
        pl.BlockSpec((1, DOUT), lambda m: (0, 0)),
    ],
    out_specs=[
        pl.BlockSpec((BM, DOUT), lambda m: (m, 0)),
        pl.BlockSpec((BM, DOUT), lambda m: (m, 0)),
    ],
    out_shape=[
        jax.ShapeDtypeStruct((N, DOUT), jnp.float32),
        jax.ShapeDtypeStruct((N, DOUT), jnp.float32),
    ],
)


def kernel(x, edge_index, W1, b1, W2, b2, LW1, Lb1, LW2, Lb2):
    f32 = jnp.float32
    edge_p = jnp.pad(edge_index, ((0, 0), (0, EPAD - E)), constant_values=N)
    Wc = jnp.concatenate([W1, W2], axis=1)
    bc = jnp.concatenate([b1, b2])[None, :]
    LW1p = jnp.zeros((F, DOUT), f32).at[0:DH].set(LW1)
    LW2p = jnp.zeros((F, DOUT), f32).at[DH:2 * DH].set(LW2)
    ones = jnp.ones((CHUNK,), f32)
    zeros1 = jnp.zeros((NPAD,), f32)
    zeros2 = jnp.zeros((NPAD, F), f32)

    xw = _tc_xw_call(x, Wc)
    cnt = _sc_degree_call()(edge_p, ones, zeros1)
    agg = _sc_aggregate_call()(edge_p, xw, cnt, zeros2)
    o1, o2 = _tc_heads_call(xw, agg, cnt.T, bc, LW1p, Lb1[None], LW2p,
                            Lb2[None])
    return (o1, o2)


# speedup vs baseline: 68.1669x; 1.1860x over previous
import functools

import jax
import jax.numpy as jnp
from jax import lax
from jax.experimental import pallas as pl
from jax.experimental.pallas import tpu as pltpu
from jax.experimental.pallas import tpu_sc as plsc

N = 10000
NPAD = 10240
E = 320000
DIN = 128
DH = 20
DOUT = 128
F = 40
NC, NS = 2, 16
NTILES = NC * NS
CHUNK = 128
EPT = 10240
EPAD = NTILES * EPT
NCHUNK = EPT // CHUNK
NPAIR = NCHUNK // 2
SLICE = NPAD // NS


def _sc_degree(edge_hbm, ones_hbm, zeros_hbm, cnt_hbm, idx0, idx1, ones_v,
               stage_v, si0, si1, ss0, ss1, cnt_sh):
    c = lax.axis_index("c")
    s = lax.axis_index("s")
    tid = c * NS + s
    ebase = tid * EPT
    idx = (idx0, idx1)
    si = (si0, si1)
    ss = (ss0, ss1)
    pltpu.sync_copy(zeros_hbm.at[pl.ds(s * SLICE, SLICE)], stage_v)
    pltpu.sync_copy(stage_v, cnt_sh.at[pl.ds(s * SLICE, SLICE)])
    pltpu.sync_copy(ones_hbm, ones_v)
    plsc.subcore_barrier()

    for b in (0, 1):
        pltpu.async_copy(edge_hbm.at[1, pl.ds(ebase + b * CHUNK, CHUNK)],
                         idx[b], si[b])

    def chunk(i, b, prefetch):
        pltpu.make_async_copy(edge_hbm.at[1, pl.ds(0, CHUNK)], idx[b],
                              si[b]).wait()
        sc = pltpu.async_copy(ones_v, cnt_sh.at[idx[b]], ss[b], add=True)
        sc.wait()
        if prefetch:
            pltpu.async_copy(
                edge_hbm.at[1, pl.ds(ebase + (i + 2) * CHUNK, CHUNK)], idx[b],
                si[b])

    def body(p, carry):
        for b in (0, 1):
            chunk(2 * p + b, b, True)
        return carry

    lax.fori_loop(0, NPAIR - 1, body, 0)
    for b in (0, 1):
        chunk(NCHUNK - 2 + b, b, False)
    plsc.subcore_barrier()
    pltpu.sync_copy(cnt_sh.at[pl.ds(s * SLICE, SLICE)], stage_v)
    pltpu.sync_copy(stage_v, cnt_hbm.at[c, pl.ds(s * SLICE, SLICE)])


@functools.cache
def _sc_degree_call():
    mesh = plsc.VectorSubcoreMesh(core_axis_name="c", subcore_axis_name="s",
                                  num_cores=NC, num_subcores=NS)
    return pl.kernel(
        _sc_degree,
        out_type=jax.ShapeDtypeStruct((NC, NPAD), jnp.float32),
        mesh=mesh,
        scratch_types=[
            pltpu.VMEM((CHUNK,), jnp.int32),
            pltpu.VMEM((CHUNK,), jnp.int32),
            pltpu.VMEM((CHUNK,), jnp.float32),
            pltpu.VMEM((SLICE,), jnp.float32),
            pltpu.SemaphoreType.DMA,
            pltpu.SemaphoreType.DMA,
            pltpu.SemaphoreType.DMA,
            pltpu.SemaphoreType.DMA,
            pltpu.VMEM_SHARED((NPAD,), jnp.float32),
        ],
        compiler_params=pltpu.CompilerParams(use_tc_tiling_on_sc=False),
    )


def _sc_aggregate(edge_hbm, xw_hbm, cnt_hbm, zeros_hbm, agg_hbm, isrc0, isrc1,
                  isrc2, isrc3, idst0, idst1, idst2, idst3, rows0, rows1,
                  rows2, rows3, stage_v, stage2_v, c0_v, c1_v, dis_v,
                  sis0, sis1, sis2, sis3, sid0, sid1, sid2, sid3,
                  sg0, sg1, sg2, sg3, ss0, ss1, ss2, ss3, agg_sh, y_sh):
    c = lax.axis_index("c")
    s = lax.axis_index("s")
    tid = c * NS + s
    ebase = tid * EPT
    isrc = (isrc0, isrc1, isrc2, isrc3)
    idst = (idst0, idst1, idst2, idst3)
    rows = (rows0, rows1, rows2, rows3)
    sis = (sis0, sis1, sis2, sis3)
    sid = (sid0, sid1, sid2, sid3)
    sg = (sg0, sg1, sg2, sg3)
    ss = (ss0, ss1, ss2, ss3)
    pltpu.sync_copy(zeros_hbm.at[pl.ds(s * SLICE, SLICE)], stage_v)
    pltpu.sync_copy(stage_v, agg_sh.at[pl.ds(s * SLICE, SLICE)])
    pltpu.sync_copy(xw_hbm.at[pl.ds(s * SLICE, SLICE)], stage_v)
    pltpu.sync_copy(cnt_hbm.at[0, pl.ds(s * SLICE, SLICE)], c0_v)
    pltpu.sync_copy(cnt_hbm.at[1, pl.ds(s * SLICE, SLICE)], c1_v)

    def disbody(k, carry):
        d = 1.0 + c0_v[pl.ds(16 * k, 16)] + c1_v[pl.ds(16 * k, 16)]
        g = plsc.bitcast(0x5F3759DF - (plsc.bitcast(d, jnp.int32) >> 1),
                         jnp.float32)
        g = g * (1.5 - 0.5 * d * g * g)
        g = g * (1.5 - 0.5 * d * g * g)
        g = g * (1.5 - 0.5 * d * g * g)
        dis_v[pl.ds(16 * k, 16)] = g
        return carry

    lax.fori_loop(0, SLICE // 16, disbody, 0)

    def rowblock(k, carry):
        dv = dis_v[pl.ds(16 * k, 16)]
        for j in range(16):
            r = 16 * k + j
            ga = dv[j]
            stage2_v[r, pl.ds(0, 16)] = stage_v[r, pl.ds(0, 16)] * ga
            stage2_v[r, pl.ds(16, 16)] = stage_v[r, pl.ds(16, 16)] * ga
            stage2_v[r, pl.ds(24, 16)] = stage_v[r, pl.ds(24, 16)] * ga
        return carry

    lax.fori_loop(0, SLICE // 16, rowblock, 0)
    pltpu.sync_copy(stage2_v, y_sh.at[pl.ds(s * SLICE, SLICE)])
    plsc.subcore_barrier()

    def fetch_idx(j, t):
        pltpu.async_copy(edge_hbm.at[0, pl.ds(ebase + j * CHUNK, CHUNK)],
                         isrc[t], sis[t])
        pltpu.async_copy(edge_hbm.at[1, pl.ds(ebase + j * CHUNK, CHUNK)],
                         idst[t], sid[t])

    def fire_gather(t):
        pltpu.make_async_copy(edge_hbm.at[0, pl.ds(0, CHUNK)], isrc[t],
                              sis[t]).wait()
        pltpu.async_copy(y_sh.at[isrc[t]], rows[t], sg[t])

    def do_scatter(t):
        pltpu.make_async_copy(y_sh.at[pl.ds(0, CHUNK)], rows[t],
                              sg[t]).wait()
        pltpu.make_async_copy(edge_hbm.at[1, pl.ds(0, CHUNK)], idst[t],
                              sid[t]).wait()
        pltpu.async_copy(rows[t], agg_sh.at[idst[t]], ss[t], add=True)

    def wait_scatter(t):
        pltpu.make_async_copy(rows[t], agg_sh.at[pl.ds(0, CHUNK)],
                              ss[t]).wait()

    fetch_idx(0, 0)
    fetch_idx(1, 1)
    fire_gather(0)
    do_scatter(0)
    fetch_idx(2, 2)
    fire_gather(1)
    do_scatter(1)
    fetch_idx(3, 3)
    fire_gather(2)

    def body(p, carry):
        for j in range(4):
            i = 2 + 4 * p + j
            b = (2 + j) % 4
            do_scatter(b)
            f = (j + 4) % 4
            wait_scatter(f)
            fetch_idx(i + 2, f)
            fire_gather((3 + j) % 4)
        return carry

    lax.fori_loop(0, (NCHUNK - 4) // 4, body, 0)
    do_scatter(2)
    fire_gather(3)
    do_scatter(3)
    for t in range(4):
        wait_scatter(t)
    plsc.subcore_barrier()
    pltpu.sync_copy(agg_sh.at[pl.ds(s * SLICE, SLICE)], stage_v)
    pltpu.sync_copy(stage_v, agg_hbm.at[c, pl.ds(s * SLICE, SLICE)])


@functools.cache
def _sc_aggregate_call():
    mesh = plsc.VectorSubcoreMesh(core_axis_name="c", subcore_axis_name="s",
                                  num_cores=NC, num_subcores=NS)
    return pl.kernel(
        _sc_aggregate,
        out_type=jax.ShapeDtypeStruct((NC, NPAD, F), jnp.float32),
        mesh=mesh,
        scratch_types=(
            [pltpu.VMEM((CHUNK,), jnp.int32)] * 8
            + [pltpu.VMEM((CHUNK, F), jnp.float32)] * 4
            + [pltpu.VMEM((SLICE, F), jnp.float32)] * 2
            + [pltpu.VMEM((SLICE,), jnp.float32)] * 3
            + [pltpu.SemaphoreType.DMA] * 16
            + [pltpu.VMEM_SHARED((NPAD, F), jnp.float32)] * 2
        ),
        compiler_params=pltpu.CompilerParams(use_tc_tiling_on_sc=False,
                                             needs_layout_passes=False),
    )


BM = 2000


def _tc_xw_body(x_ref, w_ref, xw_ref):
    xw_ref[...] = jnp.dot(x_ref[...], w_ref[...],
                          preferred_element_type=jnp.float32)


_tc_xw_call = pl.pallas_call(
    _tc_xw_body,
    grid=(N // BM,),
    in_specs=[
        pl.BlockSpec((BM, DIN), lambda m: (m, 0)),
        pl.BlockSpec((DIN, F), lambda m: (0, 0)),
    ],
    out_specs=pl.BlockSpec((BM, F), lambda m: (m, 0)),
    out_shape=jax.ShapeDtypeStruct((NPAD, F), jnp.float32),
)


def _tc_heads_body(xw_ref, agg_ref, cntT_ref, bc_ref, lw1_ref, lb1_ref,
                   lw2_ref, lb2_ref, o1_ref, o2_ref):
    deg = 1.0 + cntT_ref[:, 0:1] + cntT_ref[:, 1:2]
    dis = lax.rsqrt(deg)
    feat = dis * (agg_ref[0] + agg_ref[1]) + (dis * dis) * xw_ref[...] \
        + bc_ref[...]
    h1 = jnp.dot(feat, lw1_ref[...],
                 preferred_element_type=jnp.float32) + lb1_ref[...]
    h2 = jnp.dot(feat, lw2_ref[...],
                 preferred_element_type=jnp.float32) + lb2_ref[...]
    n1 = jnp.sqrt(jnp.sum(h1 * h1, axis=1, keepdims=True))
    n2 = jnp.sqrt(jnp.sum(h2 * h2, axis=1, keepdims=True))
    o1_ref[...] = h1 / jnp.maximum(n1, 1e-12)
    o2_ref[...] = h2 / jnp.maximum(n2, 1e-12)


_tc_heads_call = pl.pallas_call(
    _tc_heads_body,
    grid=(N // BM,),
    in_specs=[
        pl.BlockSpec((BM, F), lambda m: (m, 0)),
        pl.BlockSpec((NC, BM, F), lambda m: (0, m, 0)),
        pl.BlockSpec((BM, NC), lambda m: (m, 0)),
        pl.BlockSpec((1, F), lambda m: (0, 0)),
        pl.BlockSpec((F, DOUT), lambda m: (0, 0)),
        pl.BlockSpec((1, DOUT), lambda m: (0, 0)),
        pl.BlockSpec((F, DOUT), lambda m: (0, 0)),
        pl.BlockSpec((1, DOUT), lambda m: (0, 0)),
    ],
    out_specs=[
        pl.BlockSpec((BM, DOUT), lambda m: (m, 0)),
        pl.BlockSpec((BM, DOUT), lambda m: (m, 0)),
    ],
    out_shape=[
        jax.ShapeDtypeStruct((N, DOUT), jnp.float32),
        jax.ShapeDtypeStruct((N, DOUT), jnp.float32),
    ],
)


def kernel(x, edge_index, W1, b1, W2, b2, LW1, Lb1, LW2, Lb2):
    f32 = jnp.float32
    edge_p = jnp.pad(edge_index, ((0, 0), (0, EPAD - E)), constant_values=N)
    Wc = jnp.concatenate([W1, W2], axis=1)
    bc = jnp.concatenate([b1, b2])[None, :]
    LW1p = jnp.zeros((F, DOUT), f32).at[0:DH].set(LW1)
    LW2p = jnp.zeros((F, DOUT), f32).at[DH:2 * DH].set(LW2)
    ones = jnp.ones((CHUNK,), f32)
    zeros1 = jnp.zeros((NPAD,), f32)
    zeros2 = jnp.zeros((NPAD, F), f32)

    xw = _tc_xw_call(x, Wc)
    cnt = _sc_degree_call()(edge_p, ones, zeros1)
    agg = _sc_aggregate_call()(edge_p, xw, cnt, zeros2)
    o1, o2 = _tc_heads_call(xw, agg, cnt.T, bc, LW1p, Lb1[None], LW2p,
                            Lb2[None])
    return (o1, o2)


# speedup vs baseline: 69.8651x; 1.0249x over previous
import functools

import jax
import jax.numpy as jnp
from jax import lax
from jax.experimental import pallas as pl
from jax.experimental.pallas import tpu as pltpu
from jax.experimental.pallas import tpu_sc as plsc

N = 10000
NPAD = 10240
E = 320000
DIN = 128
DH = 20
DOUT = 128
F = 40
NC, NS = 2, 16
NTILES = NC * NS
CHUNK = 128
EPT = 10240
EPAD = NTILES * EPT
NCHUNK = EPT // CHUNK
NPAIR = NCHUNK // 2
SLICE = NPAD // NS


def _sc_degree(edge_hbm, ones_hbm, zeros_hbm, cnt_hbm, idx0, idx1, idx2, idx3,
               ones_v, stage_v, si0, si1, si2, si3, ss0, ss1, ss2, ss3,
               cnt_sh):
    c = lax.axis_index("c")
    s = lax.axis_index("s")
    tid = c * NS + s
    ebase = tid * EPT
    idx = (idx0, idx1, idx2, idx3)
    si = (si0, si1, si2, si3)
    ss = (ss0, ss1, ss2, ss3)
    pltpu.sync_copy(zeros_hbm.at[pl.ds(s * SLICE, SLICE)], stage_v)
    pltpu.sync_copy(stage_v, cnt_sh.at[pl.ds(s * SLICE, SLICE)])
    pltpu.sync_copy(ones_hbm, ones_v)
    plsc.subcore_barrier()

    def fetch_idx(j, t):
        pltpu.async_copy(edge_hbm.at[1, pl.ds(ebase + j * CHUNK, CHUNK)],
                         idx[t], si[t])

    def do_scatter(t):
        pltpu.make_async_copy(edge_hbm.at[1, pl.ds(0, CHUNK)], idx[t],
                              si[t]).wait()
        pltpu.async_copy(ones_v, cnt_sh.at[idx[t]], ss[t], add=True)

    def wait_scatter(t):
        pltpu.make_async_copy(ones_v, cnt_sh.at[pl.ds(0, CHUNK)],
                              ss[t]).wait()

    fetch_idx(0, 0)
    fetch_idx(1, 1)
    do_scatter(0)
    fetch_idx(2, 2)
    do_scatter(1)
    fetch_idx(3, 3)

    def body(p, carry):
        for j in range(4):
            i = 2 + 4 * p + j
            do_scatter((2 + j) % 4)
            wait_scatter(j)
            fetch_idx(i + 2, j)
        return carry

    lax.fori_loop(0, (NCHUNK - 4) // 4, body, 0)
    do_scatter(2)
    do_scatter(3)
    for t in range(4):
        wait_scatter(t)
    plsc.subcore_barrier()
    pltpu.sync_copy(cnt_sh.at[pl.ds(s * SLICE, SLICE)], stage_v)
    pltpu.sync_copy(stage_v, cnt_hbm.at[c, pl.ds(s * SLICE, SLICE)])


@functools.cache
def _sc_degree_call():
    mesh = plsc.VectorSubcoreMesh(core_axis_name="c", subcore_axis_name="s",
                                  num_cores=NC, num_subcores=NS)
    return pl.kernel(
        _sc_degree,
        out_type=jax.ShapeDtypeStruct((NC, NPAD), jnp.float32),
        mesh=mesh,
        scratch_types=(
            [pltpu.VMEM((CHUNK,), jnp.int32)] * 4
            + [pltpu.VMEM((CHUNK,), jnp.float32)]
            + [pltpu.VMEM((SLICE,), jnp.float32)]
            + [pltpu.SemaphoreType.DMA] * 8
            + [pltpu.VMEM_SHARED((NPAD,), jnp.float32)]
        ),
        compiler_params=pltpu.CompilerParams(use_tc_tiling_on_sc=False),
    )


def _sc_aggregate(edge_hbm, xw_hbm, cnt_hbm, zeros_hbm, agg_hbm, isrc0, isrc1,
                  isrc2, isrc3, idst0, idst1, idst2, idst3, rows0, rows1,
                  rows2, rows3, stage_v, stage2_v, c0_v, c1_v, dis_v,
                  sis0, sis1, sis2, sis3, sid0, sid1, sid2, sid3,
                  sg0, sg1, sg2, sg3, ss0, ss1, ss2, ss3, agg_sh, y_sh):
    c = lax.axis_index("c")
    s = lax.axis_index("s")
    tid = c * NS + s
    ebase = tid * EPT
    isrc = (isrc0, isrc1, isrc2, isrc3)
    idst = (idst0, idst1, idst2, idst3)
    rows = (rows0, rows1, rows2, rows3)
    sis = (sis0, sis1, sis2, sis3)
    sid = (sid0, sid1, sid2, sid3)
    sg = (sg0, sg1, sg2, sg3)
    ss = (ss0, ss1, ss2, ss3)
    pltpu.sync_copy(zeros_hbm.at[pl.ds(s * SLICE, SLICE)], stage_v)
    pltpu.sync_copy(stage_v, agg_sh.at[pl.ds(s * SLICE, SLICE)])
    pltpu.sync_copy(xw_hbm.at[pl.ds(s * SLICE, SLICE)], stage_v)
    pltpu.sync_copy(cnt_hbm.at[0, pl.ds(s * SLICE, SLICE)], c0_v)
    pltpu.sync_copy(cnt_hbm.at[1, pl.ds(s * SLICE, SLICE)], c1_v)

    def disbody(k, carry):
        d = 1.0 + c0_v[pl.ds(16 * k, 16)] + c1_v[pl.ds(16 * k, 16)]
        g = plsc.bitcast(0x5F3759DF - (plsc.bitcast(d, jnp.int32) >> 1),
                         jnp.float32)
        g = g * (1.5 - 0.5 * d * g * g)
        g = g * (1.5 - 0.5 * d * g * g)
        g = g * (1.5 - 0.5 * d * g * g)
        dis_v[pl.ds(16 * k, 16)] = g
        return carry

    lax.fori_loop(0, SLICE // 16, disbody, 0)

    def rowblock(k, carry):
        dv = dis_v[pl.ds(16 * k, 16)]
        for j in range(16):
            r = 16 * k + j
            ga = dv[j]
            stage2_v[r, pl.ds(0, 16)] = stage_v[r, pl.ds(0, 16)] * ga
            stage2_v[r, pl.ds(16, 16)] = stage_v[r, pl.ds(16, 16)] * ga
            stage2_v[r, pl.ds(24, 16)] = stage_v[r, pl.ds(24, 16)] * ga
        return carry

    lax.fori_loop(0, SLICE // 16, rowblock, 0)
    pltpu.sync_copy(stage2_v, y_sh.at[pl.ds(s * SLICE, SLICE)])
    plsc.subcore_barrier()

    def fetch_idx(j, t):
        pltpu.async_copy(edge_hbm.at[0, pl.ds(ebase + j * CHUNK, CHUNK)],
                         isrc[t], sis[t])
        pltpu.async_copy(edge_hbm.at[1, pl.ds(ebase + j * CHUNK, CHUNK)],
                         idst[t], sid[t])

    def fire_gather(t):
        pltpu.make_async_copy(edge_hbm.at[0, pl.ds(0, CHUNK)], isrc[t],
                              sis[t]).wait()
        pltpu.async_copy(y_sh.at[isrc[t]], rows[t], sg[t])

    def do_scatter(t):
        pltpu.make_async_copy(y_sh.at[pl.ds(0, CHUNK)], rows[t],
                              sg[t]).wait()
        pltpu.make_async_copy(edge_hbm.at[1, pl.ds(0, CHUNK)], idst[t],
                              sid[t]).wait()
        pltpu.async_copy(rows[t], agg_sh.at[idst[t]], ss[t], add=True)

    def wait_scatter(t):
        pltpu.make_async_copy(rows[t], agg_sh.at[pl.ds(0, CHUNK)],
                              ss[t]).wait()

    fetch_idx(0, 0)
    fetch_idx(1, 1)
    fire_gather(0)
    do_scatter(0)
    fetch_idx(2, 2)
    fire_gather(1)
    do_scatter(1)
    fetch_idx(3, 3)
    fire_gather(2)

    def body(p, carry):
        for j in range(4):
            i = 2 + 4 * p + j
            b = (2 + j) % 4
            do_scatter(b)
            f = (j + 4) % 4
            wait_scatter(f)
            fetch_idx(i + 2, f)
            fire_gather((3 + j) % 4)
        return carry

    lax.fori_loop(0, (NCHUNK - 4) // 4, body, 0)
    do_scatter(2)
    fire_gather(3)
    do_scatter(3)
    for t in range(4):
        wait_scatter(t)
    plsc.subcore_barrier()
    pltpu.sync_copy(agg_sh.at[pl.ds(s * SLICE, SLICE)], stage_v)
    pltpu.sync_copy(stage_v, agg_hbm.at[c, pl.ds(s * SLICE, SLICE)])


@functools.cache
def _sc_aggregate_call():
    mesh = plsc.VectorSubcoreMesh(core_axis_name="c", subcore_axis_name="s",
                                  num_cores=NC, num_subcores=NS)
    return pl.kernel(
        _sc_aggregate,
        out_type=jax.ShapeDtypeStruct((NC, NPAD, F), jnp.float32),
        mesh=mesh,
        scratch_types=(
            [pltpu.VMEM((CHUNK,), jnp.int32)] * 8
            + [pltpu.VMEM((CHUNK, F), jnp.float32)] * 4
            + [pltpu.VMEM((SLICE, F), jnp.float32)] * 2
            + [pltpu.VMEM((SLICE,), jnp.float32)] * 3
            + [pltpu.SemaphoreType.DMA] * 16
            + [pltpu.VMEM_SHARED((NPAD, F), jnp.float32)] * 2
        ),
        compiler_params=pltpu.CompilerParams(use_tc_tiling_on_sc=False,
                                             needs_layout_passes=False),
    )


BM = 2000


def _tc_xw_body(x_ref, w_ref, xw_ref):
    xw_ref[...] = jnp.dot(x_ref[...], w_ref[...],
                          preferred_element_type=jnp.float32)


_tc_xw_call = pl.pallas_call(
    _tc_xw_body,
    grid=(N // BM,),
    in_specs=[
        pl.BlockSpec((BM, DIN), lambda m: (m, 0)),
        pl.BlockSpec((DIN, F), lambda m: (0, 0)),
    ],
    out_specs=pl.BlockSpec((BM, F), lambda m: (m, 0)),
    out_shape=jax.ShapeDtypeStruct((NPAD, F), jnp.float32),
)


def _tc_heads_body(xw_ref, agg_ref, cntT_ref, bc_ref, lw1_ref, lb1_ref,
                   lw2_ref, lb2_ref, o1_ref, o2_ref):
    deg = 1.0 + cntT_ref[:, 0:1] + cntT_ref[:, 1:2]
    dis = lax.rsqrt(deg)
    feat = dis * (agg_ref[0] + agg_ref[1]) + (dis * dis) * xw_ref[...] \
        + bc_ref[...]
    h1 = jnp.dot(feat, lw1_ref[...],
                 preferred_element_type=jnp.float32) + lb1_ref[...]
    h2 = jnp.dot(feat, lw2_ref[...],
                 preferred_element_type=jnp.float32) + lb2_ref[...]
    n1 = jnp.sqrt(jnp.sum(h1 * h1, axis=1, keepdims=True))
    n2 = jnp.sqrt(jnp.sum(h2 * h2, axis=1, keepdims=True))
    o1_ref[...] = h1 / jnp.maximum(n1, 1e-12)
    o2_ref[...] = h2 / jnp.maximum(n2, 1e-12)


_tc_heads_call = pl.pallas_call(
    _tc_heads_body,
    grid=(N // BM,),
    in_specs=[
        pl.BlockSpec((BM, F), lambda m: (m, 0)),
        pl.BlockSpec((NC, BM, F), lambda m: (0, m, 0)),
        pl.BlockSpec((BM, NC), lambda m: (m, 0)),
        pl.BlockSpec((1, F), lambda m: (0, 0)),
        pl.BlockSpec((F, DOUT), lambda m: (0, 0)),
        pl.BlockSpec((1, DOUT), lambda m: (0, 0)),
        pl.BlockSpec((F, DOUT), lambda m: (0, 0)),
        pl.BlockSpec((1, DOUT), lambda m: (0, 0)),
    ],
    out_specs=[
        pl.BlockSpec((BM, DOUT), lambda m: (m, 0)),
        pl.BlockSpec((BM, DOUT), lambda m: (m, 0)),
    ],
    out_shape=[
        jax.ShapeDtypeStruct((N, DOUT), jnp.float32),
        jax.ShapeDtypeStruct((N, DOUT), jnp.float32),
    ],
)


def kernel(x, edge_index, W1, b1, W2, b2, LW1, Lb1, LW2, Lb2):
    f32 = jnp.float32
    edge_p = jnp.pad(edge_index, ((0, 0), (0, EPAD - E)), constant_values=N)
    Wc = jnp.concatenate([W1, W2], axis=1)
    bc = jnp.concatenate([b1, b2])[None, :]
    LW1p = jnp.zeros((F, DOUT), f32).at[0:DH].set(LW1)
    LW2p = jnp.zeros((F, DOUT), f32).at[DH:2 * DH].set(LW2)
    ones = jnp.ones((CHUNK,), f32)
    zeros1 = jnp.zeros((NPAD,), f32)
    zeros2 = jnp.zeros((NPAD, F), f32)

    xw = _tc_xw_call(x, Wc)
    cnt = _sc_degree_call()(edge_p, ones, zeros1)
    agg = _sc_aggregate_call()(edge_p, xw, cnt, zeros2)
    o1, o2 = _tc_heads_call(xw, agg, cnt.T, bc, LW1p, Lb1[None], LW2p,
                            Lb2[None])
    return (o1, o2)
